# baseline (speedup 1.0000x reference)
import functools

import jax
import jax.numpy as jnp
from jax import lax
from jax.experimental import pallas as pl
from jax.experimental.pallas import tpu as pltpu
from jax.experimental.pallas import tpu_sc as plsc

B = 16384
D_NODE, D_FIN, D_NFIN, D_BERT = 128, 84, 11, 768
PAD = 128
D_HID = 1982
NC, NS = 2, 16
NW = NC * NS
BPW = B // NW
CH = 128
CHB = 64
EPS = 1e-4


K = 16


def _gather_body(node_idx, fin_idx, nfin_idx, mda_idx,
                 node_table, fin_table, nfin_table, bert_table,
                 out_node, out_fin, out_nfin, out_bert,
                 idx_n, idx_m, idx_t,
                 buf_n, buf_b, sem, semr):
    wid = lax.axis_index("s") * NC + lax.axis_index("c")
    base = wid * BPW
    pltpu.sync_copy(node_idx.at[pl.ds(base, BPW)], idx_n)
    pltpu.sync_copy(mda_idx.at[pl.ds(base, BPW)], idx_m)

    lanes = lax.broadcasted_iota(jnp.int32, (16,), 0)

    def rowwise(idx_hbm, table, out):
        pltpu.sync_copy(idx_hbm.at[pl.ds(base, BPW)], idx_t)

        @pl.loop(0, BPW // K)
        def _(c):
            v = idx_t[pl.ds(c * K, K)]
            cps = []
            for k in range(K):
                i = jnp.sum(jnp.where(lanes == k, v, 0))
                cps.append(pltpu.async_copy(
                    table.at[pl.ds(i, 1)],
                    out.at[pl.ds(base + c * K + k, 1)], semr))
            for cp in cps:
                cp.wait()

    rowwise(fin_idx, fin_table, out_fin)
    rowwise(nfin_idx, nfin_table, out_nfin)

    for c in range(BPW // CH):
        pltpu.async_copy(
            node_table.at[idx_n.at[pl.ds(c * CH, CH)]], buf_n, sem).wait()
        pltpu.sync_copy(buf_n, out_node.at[pl.ds(base + c * CH, CH)])
    for c in range(BPW // CHB):
        pltpu.async_copy(
            bert_table.at[idx_m.at[pl.ds(c * CHB, CHB)]], buf_b, sem).wait()
        pltpu.sync_copy(buf_b, out_bert.at[pl.ds(base + c * CHB, CHB)])


@functools.cache
def _build_gather():
    mesh = plsc.VectorSubcoreMesh(core_axis_name="c", subcore_axis_name="s",
                                  num_cores=NC, num_subcores=NS)
    return pl.kernel(
        _gather_body,
        out_type=(
            jax.ShapeDtypeStruct((B, D_NODE), jnp.float32),
            jax.ShapeDtypeStruct((B, D_FIN), jnp.float32),
            jax.ShapeDtypeStruct((B, D_NFIN), jnp.float32),
            jax.ShapeDtypeStruct((B, D_BERT), jnp.float32),
        ),
        mesh=mesh,
        scratch_types=[
            pltpu.VMEM((BPW,), jnp.int32),
            pltpu.VMEM((BPW,), jnp.int32),
            pltpu.VMEM((BPW,), jnp.int32),
            pltpu.VMEM((CH, D_NODE), jnp.float32),
            pltpu.VMEM((CHB, D_BERT), jnp.float32),
            pltpu.SemaphoreType.DMA,
            pltpu.SemaphoreType.DMA,
        ],
        compiler_params=pltpu.CompilerParams(needs_layout_passes=False),
    )


ROWS = 512
WIDTHS = (D_NODE, D_FIN, D_NFIN, D_BERT)


def _stats_kernel(xn, xf, xnf, xb, on, of, onf, ob):
    @pl.when(pl.program_id(0) == 0)
    def _():
        on[...] = jnp.zeros_like(on)
        of[...] = jnp.zeros_like(of)
        onf[...] = jnp.zeros_like(onf)
        ob[...] = jnp.zeros_like(ob)

    for x, o in ((xn, on), (xf, of), (xnf, onf), (xb, ob)):
        v = x[...]
        s = jnp.sum(v, axis=0, keepdims=True)
        q = jnp.sum(v * v, axis=0, keepdims=True)
        o[...] += jnp.concatenate([s, q], axis=0)


def _stats(xn, xf, xnf, xb):
    grid = (B // ROWS,)
    blk = lambda w: pl.BlockSpec((ROWS, w), lambda i: (i, 0))
    out_blk = lambda w: pl.BlockSpec((2, w), lambda i: (0, 0))
    return pl.pallas_call(
        _stats_kernel,
        grid=grid,
        in_specs=[blk(w) for w in WIDTHS],
        out_specs=tuple(out_blk(w) for w in WIDTHS),
        out_shape=tuple(
            jax.ShapeDtypeStruct((2, w), jnp.float32) for w in WIDTHS),
    )(xn, xf, xnf, xb)


def _matvec_kernel(xn, xf, xnf, xb,
                   stn, stf, stnf, stb,
                   gn, gf, gnf, gb,
                   bn, bf, bnf, bb,
                   w1n, w1f, w1nf, w1b,
                   w2, b1, b2,
                   out,
                   s_n, s_f, s_nf, s_b,
                   wc_n, wc_f, wc_nf, wc_b, cst):
    @pl.when(pl.program_id(0) == 0)
    def _():
        const = jnp.dot(b1[...], w2[...],
                        preferred_element_type=jnp.float32) + b2[...]
        for st, g, bt, w1p, s_scr, wc_scr in (
                (stn, gn, bn, w1n, s_n, wc_n),
                (stf, gf, bf, w1f, s_f, wc_f),
                (stnf, gnf, bnf, w1nf, s_nf, wc_nf),
                (stb, gb, bb, w1b, s_b, wc_b)):
            mean = st[0:1, :] * (1.0 / B)
            ex2 = st[1:2, :] * (1.0 / B)
            var = ex2 - mean * mean
            inv = lax.rsqrt(var + EPS)
            s = g[...] * inv
            c = bt[...] - mean * s
            wc = jnp.dot(w1p[...], w2[...],
                         preferred_element_type=jnp.float32)
            s_scr[...] = s
            wc_scr[...] = wc
            const = const + jnp.dot(c, wc,
                                    preferred_element_type=jnp.float32)
        cst[...] = const

    acc = jnp.broadcast_to(cst[...], (ROWS, 2))
    for x, s_scr, wc_scr in ((xn, s_n, wc_n), (xf, s_f, wc_f),
                             (xnf, s_nf, wc_nf), (xb, s_b, wc_b)):
        acc = acc + jnp.dot(x[...] * s_scr[...], wc_scr[...],
                            preferred_element_type=jnp.float32)
    out[...] = acc


def _matvec(xn, xf, xnf, xb, stats, gamma_p, beta_p, w1_p, W2, b1, b2):
    grid = (B // ROWS,)
    blk = lambda w: pl.BlockSpec((ROWS, w), lambda i: (i, 0))
    full = lambda a: pl.BlockSpec(a.shape, lambda i: tuple(0 for _ in a.shape))
    in_specs = (
        [blk(w) for w in WIDTHS]
        + [full(s) for s in stats]
        + [full(g) for g in gamma_p]
        + [full(b) for b in beta_p]
        + [full(w) for w in w1_p]
        + [full(W2), full(b1), full(b2)]
    )
    scratch = ([pltpu.VMEM((1, w), jnp.float32) for w in WIDTHS]
               + [pltpu.VMEM((w, 2), jnp.float32) for w in WIDTHS]
               + [pltpu.VMEM((1, 2), jnp.float32)])
    return pl.pallas_call(
        _matvec_kernel,
        grid=grid,
        in_specs=in_specs,
        out_specs=pl.BlockSpec((ROWS, 2), lambda i: (i, 0)),
        out_shape=jax.ShapeDtypeStruct((B, 2), jnp.float32),
        scratch_shapes=scratch,
    )(xn, xf, xnf, xb, *stats, *gamma_p, *beta_p, *w1_p, W2, b1, b2)


def kernel(node_seq, fin_seq, nfin_seq, mda_seq, seq_len,
           node_table, fin_table, nfin_table, bert_table,
           bn_gamma, bn_beta, W1, b1, W2, b2):
    ni = node_seq.reshape(B).astype(jnp.int32)
    fi = fin_seq.reshape(B).astype(jnp.int32)
    nfi = nfin_seq.reshape(B).astype(jnp.int32)
    mi = mda_seq.reshape(B).astype(jnp.int32)

    xn, xf, xnf, xb = _build_gather()(ni, fi, nfi, mi,
                                      node_table, fin_table, nfin_table,
                                      bert_table)
    stats = _stats(xn, xf, xnf, xb)

    splits = (0, D_NODE, D_NODE + D_FIN, D_NODE + D_FIN + D_NFIN,
              D_NODE + D_FIN + D_NFIN + D_BERT)
    pieces = lambda a: tuple(a[splits[i]:splits[i + 1]] for i in range(4))
    gamma_p = tuple(p.reshape(1, -1) for p in pieces(bn_gamma))
    beta_p = tuple(p.reshape(1, -1) for p in pieces(bn_beta))
    w1_p = pieces(W1)

    return _matvec(xn, xf, xnf, xb, stats, gamma_p, beta_p, w1_p,
                   W2, b1.reshape(1, -1), b2.reshape(1, -1))


# speedup vs baseline: 2.8794x; 2.8794x over previous
import functools

import jax
import jax.numpy as jnp
from jax import lax
from jax.experimental import pallas as pl
from jax.experimental.pallas import tpu as pltpu
from jax.experimental.pallas import tpu_sc as plsc

B = 16384
D_NODE, D_FIN, D_NFIN, D_BERT = 128, 84, 11, 768
PAD = 128
V1 = 50001
NC, NS = 2, 16
NW = NC * NS
BPW = B // NW
CH = 128
CHB = 32
EPS = 1e-4


def _wide_body(node_idx, mda_idx, node_table, bert_table,
               out_node, out_bert,
               idx_n, idx_m, buf_n0, buf_n1, buf_b0, buf_b1, sem0, sem1):
    wid = lax.axis_index("s") * NC + lax.axis_index("c")
    base = wid * BPW
    pltpu.sync_copy(node_idx.at[pl.ds(base, BPW)], idx_n)
    pltpu.sync_copy(mda_idx.at[pl.ds(base, BPW)], idx_m)

    def pipeline(table, out, idx, bufs, sems, ch):
        n = BPW // ch
        cps = [None, None]
        cps[0] = pltpu.async_copy(table.at[idx.at[pl.ds(0, ch)]],
                                  bufs[0], sems[0])
        for c in range(n):
            p = c % 2
            if c + 1 < n:
                cps[(c + 1) % 2] = pltpu.async_copy(
                    table.at[idx.at[pl.ds((c + 1) * ch, ch)]],
                    bufs[(c + 1) % 2], sems[(c + 1) % 2])
            cps[p].wait()
            pltpu.sync_copy(bufs[p], out.at[pl.ds(base + c * ch, ch)])

    pipeline(node_table, out_node, idx_n, (buf_n0, buf_n1), (sem0, sem1), CH)
    pipeline(bert_table, out_bert, idx_m, (buf_b0, buf_b1), (sem0, sem1), CHB)


def _narrow_body(fin_idx, nfin_idx, fin_pad, nfin_pad,
                 out_fin, out_nfin,
                 idx_f, idx_nf, buf_f0, buf_f1, buf_nf0, buf_nf1,
                 sem0, sem1):
    wid = lax.axis_index("s") * NC + lax.axis_index("c")
    base = wid * BPW
    pltpu.sync_copy(fin_idx.at[pl.ds(base, BPW)], idx_f)
    pltpu.sync_copy(nfin_idx.at[pl.ds(base, BPW)], idx_nf)

    def pipeline(table, out, idx, bufs, sems, ch):
        n = BPW // ch
        cps = [None, None]
        cps[0] = pltpu.async_copy(table.at[idx.at[pl.ds(0, ch)]],
                                  bufs[0], sems[0])
        for c in range(n):
            p = c % 2
            if c + 1 < n:
                cps[(c + 1) % 2] = pltpu.async_copy(
                    table.at[idx.at[pl.ds((c + 1) * ch, ch)]],
                    bufs[(c + 1) % 2], sems[(c + 1) % 2])
            cps[p].wait()
            pltpu.sync_copy(bufs[p], out.at[pl.ds(base + c * ch, ch)])

    pipeline(fin_pad, out_fin, idx_f, (buf_f0, buf_f1), (sem0, sem1), CH)
    pipeline(nfin_pad, out_nfin, idx_nf, (buf_nf0, buf_nf1),
             (sem0, sem1), CH)


@functools.cache
def _build_gathers():
    mesh = plsc.VectorSubcoreMesh(core_axis_name="c", subcore_axis_name="s",
                                  num_cores=NC, num_subcores=NS)
    wide = pl.kernel(
        _wide_body,
        out_type=(
            jax.ShapeDtypeStruct((B, D_NODE), jnp.float32),
            jax.ShapeDtypeStruct((B, D_BERT), jnp.float32),
        ),
        mesh=mesh,
        scratch_types=[
            pltpu.VMEM((BPW,), jnp.int32),
            pltpu.VMEM((BPW,), jnp.int32),
            pltpu.VMEM((CH, D_NODE), jnp.float32),
            pltpu.VMEM((CH, D_NODE), jnp.float32),
            pltpu.VMEM((CHB, D_BERT), jnp.float32),
            pltpu.VMEM((CHB, D_BERT), jnp.float32),
            pltpu.SemaphoreType.DMA,
            pltpu.SemaphoreType.DMA,
        ],
    )
    narrow = pl.kernel(
        _narrow_body,
        out_type=(
            jax.ShapeDtypeStruct((B, PAD), jnp.float32),
            jax.ShapeDtypeStruct((B, PAD), jnp.float32),
        ),
        mesh=mesh,
        scratch_types=[
            pltpu.VMEM((BPW,), jnp.int32),
            pltpu.VMEM((BPW,), jnp.int32),
            pltpu.VMEM((CH, PAD), jnp.float32),
            pltpu.VMEM((CH, PAD), jnp.float32),
            pltpu.VMEM((CH, PAD), jnp.float32),
            pltpu.VMEM((CH, PAD), jnp.float32),
            pltpu.SemaphoreType.DMA,
            pltpu.SemaphoreType.DMA,
        ],
    )
    return wide, narrow


PROWS = 2048


def _pad_kernel(f_in, nf_in, f_out, nf_out):
    zf = jnp.zeros((PROWS, PAD - D_FIN), jnp.float32)
    znf = jnp.zeros((PROWS, PAD - D_NFIN), jnp.float32)
    f_out[...] = jnp.concatenate([f_in[...], zf], axis=1)
    nf_out[...] = jnp.concatenate([nf_in[...], znf], axis=1)


def _pad_tables(fin_table, nfin_table):
    grid = (pl.cdiv(V1, PROWS),)
    return pl.pallas_call(
        _pad_kernel,
        grid=grid,
        in_specs=[pl.BlockSpec((PROWS, D_FIN), lambda i: (i, 0)),
                  pl.BlockSpec((PROWS, D_NFIN), lambda i: (i, 0))],
        out_specs=(pl.BlockSpec((PROWS, PAD), lambda i: (i, 0)),
                   pl.BlockSpec((PROWS, PAD), lambda i: (i, 0))),
        out_shape=(jax.ShapeDtypeStruct((V1, PAD), jnp.float32),
                   jax.ShapeDtypeStruct((V1, PAD), jnp.float32)),
    )(fin_table, nfin_table)


ROWS = 512
WIDTHS = (D_NODE, PAD, PAD, D_BERT)


def _stats2_kernel(xa, xb, oa, ob):
    @pl.when(pl.program_id(0) == 0)
    def _():
        oa[...] = jnp.zeros_like(oa)
        ob[...] = jnp.zeros_like(ob)

    for x, o in ((xa, oa), (xb, ob)):
        v = x[...]
        s = jnp.sum(v, axis=0, keepdims=True)
        q = jnp.sum(v * v, axis=0, keepdims=True)
        o[...] += jnp.concatenate([s, q], axis=0)


def _stats2(xa, xb):
    wa, wb = xa.shape[1], xb.shape[1]
    grid = (B // ROWS,)
    return pl.pallas_call(
        _stats2_kernel,
        grid=grid,
        in_specs=[pl.BlockSpec((ROWS, wa), lambda i: (i, 0)),
                  pl.BlockSpec((ROWS, wb), lambda i: (i, 0))],
        out_specs=(pl.BlockSpec((2, wa), lambda i: (0, 0)),
                   pl.BlockSpec((2, wb), lambda i: (0, 0))),
        out_shape=(jax.ShapeDtypeStruct((2, wa), jnp.float32),
                   jax.ShapeDtypeStruct((2, wb), jnp.float32)),
    )(xa, xb)


def _matvec_kernel(xn, xf, xnf, xb,
                   stn, stf, stnf, stb,
                   gn, gf, gnf, gb,
                   bn, bf, bnf, bb,
                   w1n, w1f, w1nf, w1b,
                   w2, b1, b2,
                   out,
                   s_n, s_f, s_nf, s_b,
                   wc_n, wc_f, wc_nf, wc_b, cst):
    @pl.when(pl.program_id(0) == 0)
    def _():
        const = jnp.dot(b1[...], w2[...],
                        preferred_element_type=jnp.float32) + b2[...]
        for st, g, bt, w1p, s_scr, wc_scr in (
                (stn, gn, bn, w1n, s_n, wc_n),
                (stf, gf, bf, w1f, s_f, wc_f),
                (stnf, gnf, bnf, w1nf, s_nf, wc_nf),
                (stb, gb, bb, w1b, s_b, wc_b)):
            mean = st[0:1, :] * (1.0 / B)
            ex2 = st[1:2, :] * (1.0 / B)
            var = ex2 - mean * mean
            inv = lax.rsqrt(var + EPS)
            s = g[...] * inv
            c = bt[...] - mean * s
            wc = jnp.dot(w1p[...], w2[...],
                         preferred_element_type=jnp.float32)
            s_scr[...] = s
            wc_scr[...] = wc
            const = const + jnp.dot(c, wc,
                                    preferred_element_type=jnp.float32)
        cst[...] = const

    acc = jnp.broadcast_to(cst[...], (ROWS, 2))
    for x, s_scr, wc_scr in ((xn, s_n, wc_n), (xf, s_f, wc_f),
                             (xnf, s_nf, wc_nf), (xb, s_b, wc_b)):
        acc = acc + jnp.dot(x[...] * s_scr[...], wc_scr[...],
                            preferred_element_type=jnp.float32)
    out[...] = acc


def _matvec(xn, xf, xnf, xb, stats, gamma_p, beta_p, w1_p, W2, b1, b2):
    grid = (B // ROWS,)
    blk = lambda w: pl.BlockSpec((ROWS, w), lambda i: (i, 0))
    full = lambda a: pl.BlockSpec(a.shape, lambda i: tuple(0 for _ in a.shape))
    in_specs = (
        [blk(w) for w in WIDTHS]
        + [full(s) for s in stats]
        + [full(g) for g in gamma_p]
        + [full(b) for b in beta_p]
        + [full(w) for w in w1_p]
        + [full(W2), full(b1), full(b2)]
    )
    scratch = ([pltpu.VMEM((1, w), jnp.float32) for w in WIDTHS]
               + [pltpu.VMEM((w, 2), jnp.float32) for w in WIDTHS]
               + [pltpu.VMEM((1, 2), jnp.float32)])
    return pl.pallas_call(
        _matvec_kernel,
        grid=grid,
        in_specs=in_specs,
        out_specs=pl.BlockSpec((ROWS, 2), lambda i: (i, 0)),
        out_shape=jax.ShapeDtypeStruct((B, 2), jnp.float32),
        scratch_shapes=scratch,
    )(xn, xf, xnf, xb, *stats, *gamma_p, *beta_p, *w1_p, W2, b1, b2)


def kernel(node_seq, fin_seq, nfin_seq, mda_seq, seq_len,
           node_table, fin_table, nfin_table, bert_table,
           bn_gamma, bn_beta, W1, b1, W2, b2):
    ni = node_seq.reshape(B).astype(jnp.int32)
    fi = fin_seq.reshape(B).astype(jnp.int32)
    nfi = nfin_seq.reshape(B).astype(jnp.int32)
    mi = mda_seq.reshape(B).astype(jnp.int32)

    wide, narrow = _build_gathers()
    fin_pad, nfin_pad = _pad_tables(fin_table, nfin_table)
    xn, xb = wide(ni, mi, node_table, bert_table)
    xf, xnf = narrow(fi, nfi, fin_pad, nfin_pad)

    stn, stb = _stats2(xn, xb)
    stf, stnf = _stats2(xf, xnf)
    stats = (stn, stf, stnf, stb)

    splits = (0, D_NODE, D_NODE + D_FIN, D_NODE + D_FIN + D_NFIN,
              D_NODE + D_FIN + D_NFIN + D_BERT)
    pieces = lambda a: tuple(a[splits[i]:splits[i + 1]] for i in range(4))
    padw = lambda p, w: jnp.pad(p, ((0, w - p.shape[0]),) +
                                ((0, 0),) * (p.ndim - 1))
    gamma_p = tuple(padw(p, w).reshape(1, w)
                    for p, w in zip(pieces(bn_gamma), WIDTHS))
    beta_p = tuple(padw(p, w).reshape(1, w)
                   for p, w in zip(pieces(bn_beta), WIDTHS))
    w1_p = tuple(padw(p, w) for p, w in zip(pieces(W1), WIDTHS))

    return _matvec(xn, xf, xnf, xb, stats, gamma_p, beta_p, w1_p,
                   W2, b1.reshape(1, -1), b2.reshape(1, -1))


# speedup vs baseline: 3.2871x; 1.1416x over previous
import functools

import jax
import jax.numpy as jnp
from jax import lax
from jax.experimental import pallas as pl
from jax.experimental.pallas import tpu as pltpu
from jax.experimental.pallas import tpu_sc as plsc

B = 16384
D_NODE, D_FIN, D_NFIN, D_BERT = 128, 84, 11, 768
PAD = 128
V1 = 50001
NC, NS = 2, 16
NW = NC * NS
BPW = B // NW
CH = 128
CHB = 32
EPS = 1e-4


def _wide_body(node_idx, mda_idx, node_table, bert_table,
               out_node, out_bert,
               idx_n, idx_m, buf_n0, buf_n1, buf_b0, buf_b1, sem0, sem1):
    wid = lax.axis_index("s") * NC + lax.axis_index("c")
    base = wid * BPW
    pltpu.sync_copy(node_idx.at[pl.ds(base, BPW)], idx_n)
    pltpu.sync_copy(mda_idx.at[pl.ds(base, BPW)], idx_m)

    def pipeline(table, out, idx, bufs, sems, ch):
        n = BPW // ch
        cps = [None, None]
        cps[0] = pltpu.async_copy(table.at[idx.at[pl.ds(0, ch)]],
                                  bufs[0], sems[0])
        for c in range(n):
            p = c % 2
            if c + 1 < n:
                cps[(c + 1) % 2] = pltpu.async_copy(
                    table.at[idx.at[pl.ds((c + 1) * ch, ch)]],
                    bufs[(c + 1) % 2], sems[(c + 1) % 2])
            cps[p].wait()
            pltpu.sync_copy(bufs[p], out.at[pl.ds(base + c * ch, ch)])

    pipeline(node_table, out_node, idx_n, (buf_n0, buf_n1), (sem0, sem1), CH)
    pipeline(bert_table, out_bert, idx_m, (buf_b0, buf_b1), (sem0, sem1), CHB)


def _narrow_body(fin_idx, nfin_idx, fin_pad, nfin_pad,
                 out_fin, out_nfin,
                 idx_f, idx_nf, buf_f0, buf_f1, buf_nf0, buf_nf1,
                 sem0, sem1):
    wid = lax.axis_index("s") * NC + lax.axis_index("c")
    base = wid * BPW
    pltpu.sync_copy(fin_idx.at[pl.ds(base, BPW)], idx_f)
    pltpu.sync_copy(nfin_idx.at[pl.ds(base, BPW)], idx_nf)

    def pipeline(table, out, idx, bufs, sems, ch):
        n = BPW // ch
        cps = [None, None]
        cps[0] = pltpu.async_copy(table.at[idx.at[pl.ds(0, ch)]],
                                  bufs[0], sems[0])
        for c in range(n):
            p = c % 2
            if c + 1 < n:
                cps[(c + 1) % 2] = pltpu.async_copy(
                    table.at[idx.at[pl.ds((c + 1) * ch, ch)]],
                    bufs[(c + 1) % 2], sems[(c + 1) % 2])
            cps[p].wait()
            pltpu.sync_copy(bufs[p], out.at[pl.ds(base + c * ch, ch)])

    pipeline(fin_pad, out_fin, idx_f, (buf_f0, buf_f1), (sem0, sem1), CH)
    pipeline(nfin_pad, out_nfin, idx_nf, (buf_nf0, buf_nf1),
             (sem0, sem1), CH)


@functools.cache
def _build_gathers():
    mesh = plsc.VectorSubcoreMesh(core_axis_name="c", subcore_axis_name="s",
                                  num_cores=NC, num_subcores=NS)
    wide = pl.kernel(
        _wide_body,
        out_type=(
            jax.ShapeDtypeStruct((B, D_NODE), jnp.float32),
            jax.ShapeDtypeStruct((B, D_BERT), jnp.float32),
        ),
        mesh=mesh,
        scratch_types=[
            pltpu.VMEM((BPW,), jnp.int32),
            pltpu.VMEM((BPW,), jnp.int32),
            pltpu.VMEM((CH, D_NODE), jnp.float32),
            pltpu.VMEM((CH, D_NODE), jnp.float32),
            pltpu.VMEM((CHB, D_BERT), jnp.float32),
            pltpu.VMEM((CHB, D_BERT), jnp.float32),
            pltpu.SemaphoreType.DMA,
            pltpu.SemaphoreType.DMA,
        ],
    )
    narrow = pl.kernel(
        _narrow_body,
        out_type=(
            jax.ShapeDtypeStruct((B, PAD), jnp.float32),
            jax.ShapeDtypeStruct((B, PAD), jnp.float32),
        ),
        mesh=mesh,
        scratch_types=[
            pltpu.VMEM((BPW,), jnp.int32),
            pltpu.VMEM((BPW,), jnp.int32),
            pltpu.VMEM((CH, PAD), jnp.float32),
            pltpu.VMEM((CH, PAD), jnp.float32),
            pltpu.VMEM((CH, PAD), jnp.float32),
            pltpu.VMEM((CH, PAD), jnp.float32),
            pltpu.SemaphoreType.DMA,
            pltpu.SemaphoreType.DMA,
        ],
    )
    return wide, narrow


PROWS = 2048


def _pad_kernel(f_in, nf_in, f_out, nf_out):
    zf = jnp.zeros((PROWS, PAD - D_FIN), jnp.float32)
    znf = jnp.zeros((PROWS, PAD - D_NFIN), jnp.float32)
    f_out[...] = jnp.concatenate([f_in[...], zf], axis=1)
    nf_out[...] = jnp.concatenate([nf_in[...], znf], axis=1)


def _pad_tables(fin_table, nfin_table):
    grid = (pl.cdiv(V1, PROWS),)
    return pl.pallas_call(
        _pad_kernel,
        grid=grid,
        in_specs=[pl.BlockSpec((PROWS, D_FIN), lambda i: (i, 0)),
                  pl.BlockSpec((PROWS, D_NFIN), lambda i: (i, 0))],
        out_specs=(pl.BlockSpec((PROWS, PAD), lambda i: (i, 0)),
                   pl.BlockSpec((PROWS, PAD), lambda i: (i, 0))),
        out_shape=(jax.ShapeDtypeStruct((V1, PAD), jnp.float32),
                   jax.ShapeDtypeStruct((V1, PAD), jnp.float32)),
    )(fin_table, nfin_table)


ROWS = 2048
WIDTHS = (D_NODE, PAD, PAD, D_BERT)
SPLITS = (0, D_NODE, D_NODE + D_FIN, D_NODE + D_FIN + D_NFIN,
          D_NODE + D_FIN + D_NFIN + D_BERT)


def _stats2_kernel(xa, xb, oa, ob):
    @pl.when(pl.program_id(0) == 0)
    def _():
        oa[...] = jnp.zeros_like(oa)
        ob[...] = jnp.zeros_like(ob)

    for x, o in ((xa, oa), (xb, ob)):
        v = x[...]
        s = jnp.sum(v, axis=0, keepdims=True)
        q = jnp.sum(v * v, axis=0, keepdims=True)
        o[...] += jnp.concatenate([s, q], axis=0)


def _stats2(xa, xb):
    wa, wb = xa.shape[1], xb.shape[1]
    grid = (B // ROWS,)
    return pl.pallas_call(
        _stats2_kernel,
        grid=grid,
        in_specs=[pl.BlockSpec((ROWS, wa), lambda i: (i, 0)),
                  pl.BlockSpec((ROWS, wb), lambda i: (i, 0))],
        out_specs=(pl.BlockSpec((2, wa), lambda i: (0, 0)),
                   pl.BlockSpec((2, wb), lambda i: (0, 0))),
        out_shape=(jax.ShapeDtypeStruct((2, wa), jnp.float32),
                   jax.ShapeDtypeStruct((2, wb), jnp.float32)),
    )(xa, xb)


def _matvec_kernel(xn, xf, xnf, xb,
                   stn, stf, stnf, stb,
                   gamma, beta, w1, w2, b1, b2,
                   out,
                   s_n, s_f, s_nf, s_b,
                   wc_n, wc_f, wc_nf, wc_b, cst):
    @pl.when(pl.program_id(0) == 0)
    def _():
        const = jnp.dot(b1[...], w2[...],
                        preferred_element_type=jnp.float32) + b2[...]
        wc_full = jnp.dot(w1[...], w2[...],
                          preferred_element_type=jnp.float32)
        for p, (st, s_scr, wc_scr) in enumerate((
                (stn, s_n, wc_n), (stf, s_f, wc_f),
                (stnf, s_nf, wc_nf), (stb, s_b, wc_b))):
            lo, hi = SPLITS[p], SPLITS[p + 1]
            wreal, wpad = hi - lo, WIDTHS[p]
            g = gamma[:, lo:hi]
            bt = beta[:, lo:hi]
            wc = wc_full[lo:hi, :]
            if wpad > wreal:
                zl = jnp.zeros((1, wpad - wreal), jnp.float32)
                g = jnp.concatenate([g, zl], axis=1)
                bt = jnp.concatenate([bt, zl], axis=1)
                wc = jnp.concatenate(
                    [wc, jnp.zeros((wpad - wreal, 2), jnp.float32)], axis=0)
            mean = st[0:1, :] * (1.0 / B)
            ex2 = st[1:2, :] * (1.0 / B)
            var = ex2 - mean * mean
            inv = lax.rsqrt(var + EPS)
            s = g * inv
            c = bt - mean * s
            s_scr[...] = s
            wc_scr[...] = wc
            const = const + jnp.dot(c, wc,
                                    preferred_element_type=jnp.float32)
        cst[...] = const

    acc = jnp.broadcast_to(cst[...], (ROWS, 2))
    for x, s_scr, wc_scr in ((xn, s_n, wc_n), (xf, s_f, wc_f),
                             (xnf, s_nf, wc_nf), (xb, s_b, wc_b)):
        acc = acc + jnp.dot(x[...] * s_scr[...], wc_scr[...],
                            preferred_element_type=jnp.float32)
    out[...] = acc


def _matvec(xn, xf, xnf, xb, stats, gamma, beta, W1, W2, b1, b2):
    grid = (B // ROWS,)
    blk = lambda w: pl.BlockSpec((ROWS, w), lambda i: (i, 0))
    full = lambda a: pl.BlockSpec(a.shape, lambda i: tuple(0 for _ in a.shape))
    in_specs = (
        [blk(w) for w in WIDTHS]
        + [full(s) for s in stats]
        + [full(gamma), full(beta), full(W1), full(W2), full(b1), full(b2)]
    )
    scratch = ([pltpu.VMEM((1, w), jnp.float32) for w in WIDTHS]
               + [pltpu.VMEM((w, 2), jnp.float32) for w in WIDTHS]
               + [pltpu.VMEM((1, 2), jnp.float32)])
    return pl.pallas_call(
        _matvec_kernel,
        grid=grid,
        in_specs=in_specs,
        out_specs=pl.BlockSpec((ROWS, 2), lambda i: (i, 0)),
        out_shape=jax.ShapeDtypeStruct((B, 2), jnp.float32),
        scratch_shapes=scratch,
    )(xn, xf, xnf, xb, *stats, gamma, beta, W1, W2, b1, b2)


def kernel(node_seq, fin_seq, nfin_seq, mda_seq, seq_len,
           node_table, fin_table, nfin_table, bert_table,
           bn_gamma, bn_beta, W1, b1, W2, b2):
    ni = node_seq.reshape(B).astype(jnp.int32)
    fi = fin_seq.reshape(B).astype(jnp.int32)
    nfi = nfin_seq.reshape(B).astype(jnp.int32)
    mi = mda_seq.reshape(B).astype(jnp.int32)

    wide, narrow = _build_gathers()
    xn, xb = wide(ni, mi, node_table, bert_table)
    fin_pad, nfin_pad = _pad_tables(fin_table, nfin_table)
    xf, xnf = narrow(fi, nfi, fin_pad, nfin_pad)

    stf, stnf = _stats2(xf, xnf)
    stn, stb = _stats2(xn, xb)
    stats = (stn, stf, stnf, stb)

    return _matvec(xn, xf, xnf, xb, stats,
                   bn_gamma.reshape(1, -1), bn_beta.reshape(1, -1),
                   W1, W2, b1.reshape(1, -1), b2.reshape(1, -1))
